# baseline (speedup 1.0000x reference)
import functools
import math

import jax
import jax.numpy as jnp
from jax.experimental import pallas as pl
from jax.experimental.pallas import tpu as pltpu

_ALPHA = 0.1
_LAMBDA = 0.5
_ADJ_SCALE = 4096.0
_F8 = jnp.float8_e4m3fn


def _layer_math(l_num, hi_scaled, h0_blk, w):
    th = jnp.log(_LAMBDA / l_num + 1.0)
    sup = ((1.0 - _ALPHA) / _ADJ_SCALE) * hi_scaled + _ALPHA * h0_blk
    hn = jnp.maximum(
        th * jnp.dot(sup, w, preferred_element_type=jnp.float32)
        + (1.0 - th) * sup, 0.0)
    return hn


def _call1_kernel(x_ref, adj_ref, cw_ref, w0_ref, b0_ref,
                  adjres_ref, adjs_ref, h1_ref, h0_ref, h08_ref, *, R, KRES):
    l = pl.program_id(0)
    r = pl.program_id(1)
    rows = pl.ds(r * R, R)

    @pl.when(l == 0)
    def _fc0():
        h = jnp.maximum(
            jnp.dot(x_ref[...], w0_ref[...],
                    preferred_element_type=jnp.float32) + b0_ref[...], 0.0)
        h0_ref[rows, :] = h
        h08_ref[rows, :] = h.astype(_F8)

    @pl.when(l == 1)
    def _cast_and_layer1():
        a8 = (adj_ref[...] * _ADJ_SCALE).astype(_F8)
        adjres_ref[...] = a8[:, :KRES]
        adjs_ref[...] = a8[:, KRES:]
        hi = jnp.dot(a8, h08_ref[...], preferred_element_type=jnp.float32)
        hn = _layer_math(1.0, hi, h0_ref[rows, :], cw_ref[0])
        h1_ref[...] = hn.astype(_F8)


def _call2_kernel(x_ref, h1_ref, adjres_ref, adjs_ref, cw_ref, w0_ref,
                  b0_ref, w1_ref, b1_ref, out_ref, ha_ref, hb_ref, h0_ref,
                  *, R, L, KRES):
    l = pl.program_id(0)
    r = pl.program_id(1)
    rows = pl.ds(r * R, R)

    @pl.when(l == 0)
    def _fc0_and_seed():
        h = jnp.maximum(
            jnp.dot(x_ref[...], w0_ref[...],
                    preferred_element_type=jnp.float32) + b0_ref[...], 0.0)
        h0_ref[rows, :] = h
        ha_ref[rows, :] = h1_ref[...]

    def _conv(src_ref, dst_ref):
        lf = l.astype(jnp.float32) + 1.0
        hi = (jnp.dot(adjres_ref[rows, :], src_ref[:KRES, :],
                      preferred_element_type=jnp.float32)
              + jnp.dot(adjs_ref[...], src_ref[KRES:, :],
                        preferred_element_type=jnp.float32))
        hn = _layer_math(lf, hi, h0_ref[rows, :], cw_ref[0])
        dst_ref[rows, :] = hn.astype(_F8)

        @pl.when(l == L - 1)
        def _fc1():
            o = jnp.dot(hn, w1_ref[...],
                        preferred_element_type=jnp.float32) + b1_ref[...]
            m = jnp.max(o, axis=1, keepdims=True)
            s = o - m
            out_ref[...] = s - jnp.log(
                jnp.sum(jnp.exp(s), axis=1, keepdims=True))

    @pl.when((l > 0) & (l % 2 == 1))
    def _odd():
        _conv(ha_ref, hb_ref)

    @pl.when((l > 0) & (l % 2 == 0))
    def _even():
        _conv(hb_ref, ha_ref)


def kernel(x, adj, conv_weights, W_fc0, b_fc0, W_fc1, b_fc1):
    N, D = x.shape
    H = W_fc0.shape[1]
    L = conv_weights.shape[0]
    C = W_fc1.shape[1]
    R1 = next(c for c in (400, 200, 80, 40, 16, 8, N) if N % c == 0)
    R2 = next(c for c in (1000, 400, 200, 80, 40, 16, 8, N) if N % c == 0)
    KRES = min(2048, max(8, (N // 4) // 8 * 8))
    KS = N - KRES

    b0r = b_fc0.reshape(1, H)
    b1r = b_fc1.reshape(1, C)

    adjres, adjs, h1 = pl.pallas_call(
        functools.partial(_call1_kernel, R=R1, KRES=KRES),
        grid=(2, N // R1),
        in_specs=[
            pl.BlockSpec((R1, D), lambda l, r: (jnp.where(l == 0, r, 0), 0)),
            pl.BlockSpec((R1, N), lambda l, r: (jnp.where(l == 0, 0, r), 0)),
            pl.BlockSpec((1, H, H), lambda l, r: (0, 0, 0)),
            pl.BlockSpec((D, H), lambda l, r: (0, 0)),
            pl.BlockSpec((1, H), lambda l, r: (0, 0)),
        ],
        out_specs=[
            pl.BlockSpec((R1, KRES), lambda l, r: (jnp.where(l == 0, 0, r), 0)),
            pl.BlockSpec((R1, KS), lambda l, r: (jnp.where(l == 0, 0, r), 0)),
            pl.BlockSpec((R1, H), lambda l, r: (jnp.where(l == 0, 0, r), 0)),
        ],
        out_shape=[
            jax.ShapeDtypeStruct((N, KRES), _F8),
            jax.ShapeDtypeStruct((N, KS), _F8),
            jax.ShapeDtypeStruct((N, H), _F8),
        ],
        scratch_shapes=[
            pltpu.VMEM((N, H), jnp.float32),
            pltpu.VMEM((N, H), _F8),
        ],
        compiler_params=pltpu.CompilerParams(
            dimension_semantics=("arbitrary", "arbitrary")),
    )(x, adj, conv_weights, W_fc0, b0r)

    out = pl.pallas_call(
        functools.partial(_call2_kernel, R=R2, L=L, KRES=KRES),
        grid=(L, N // R2),
        in_specs=[
            pl.BlockSpec((R2, D), lambda l, r: (jnp.where(l == 0, r, 0), 0)),
            pl.BlockSpec((R2, H), lambda l, r: (jnp.where(l == 0, r, 0), 0)),
            pl.BlockSpec((N, KRES), lambda l, r: (0, 0)),
            pl.BlockSpec((R2, KS), lambda l, r: (jnp.where(l == 0, 0, r), 0)),
            pl.BlockSpec((1, H, H), lambda l, r: (l, 0, 0)),
            pl.BlockSpec((D, H), lambda l, r: (0, 0)),
            pl.BlockSpec((1, H), lambda l, r: (0, 0)),
            pl.BlockSpec((H, C), lambda l, r: (0, 0)),
            pl.BlockSpec((1, C), lambda l, r: (0, 0)),
        ],
        out_specs=pl.BlockSpec((R2, C), lambda l, r: (r, 0)),
        out_shape=jax.ShapeDtypeStruct((N, C), jnp.float32),
        scratch_shapes=[
            pltpu.VMEM((N, H), _F8),
            pltpu.VMEM((N, H), _F8),
            pltpu.VMEM((N, H), jnp.float32),
        ],
        compiler_params=pltpu.CompilerParams(
            dimension_semantics=("arbitrary", "arbitrary")),
    )(x, h1, adjres, adjs, conv_weights, W_fc0, b0r, W_fc1, b1r)
    return out


# speedup vs baseline: 1.0225x; 1.0225x over previous
import functools
import math

import jax
import jax.numpy as jnp
from jax.experimental import pallas as pl
from jax.experimental.pallas import tpu as pltpu

_ALPHA = 0.1
_LAMBDA = 0.5
_ADJ_SCALE = 4096.0
_F8 = jnp.float8_e4m3fn


def _layer_math(l_num, hi_scaled, h0_blk, w):
    th = jnp.log(_LAMBDA / l_num + 1.0)
    sup = ((1.0 - _ALPHA) / _ADJ_SCALE) * hi_scaled + _ALPHA * h0_blk
    hn = jnp.maximum(
        th * jnp.dot(sup, w, preferred_element_type=jnp.float32)
        + (1.0 - th) * sup, 0.0)
    return hn


def _call1_kernel(x_ref, adj_ref, cw_ref, w0_ref, b0_ref,
                  adj8_ref, h1_ref, h0_ref, h08_ref, *, R):
    l = pl.program_id(0)
    r = pl.program_id(1)
    rows = pl.ds(r * R, R)

    @pl.when(l == 0)
    def _fc0():
        h = jnp.maximum(
            jnp.dot(x_ref[...], w0_ref[...],
                    preferred_element_type=jnp.float32) + b0_ref[...], 0.0)
        h0_ref[rows, :] = h
        h08_ref[rows, :] = h.astype(jnp.bfloat16)

    @pl.when(l == 1)
    def _cast_and_layer1():
        a32 = adj_ref[...] * _ADJ_SCALE
        adj8_ref[...] = a32.astype(_F8)
        hi = jnp.dot(a32.astype(jnp.bfloat16), h08_ref[...],
                     preferred_element_type=jnp.float32)
        hn = _layer_math(1.0, hi, h0_ref[rows, :], cw_ref[0])
        h1_ref[...] = hn.astype(_F8)


def _call2_kernel(x_ref, h1_ref, adj8_ref, cw_ref, w0_ref, b0_ref,
                  w1_ref, b1_ref, out_ref, ha_ref, hb_ref, h0_ref, *, R, L):
    l = pl.program_id(0)
    r = pl.program_id(1)
    rows = pl.ds(r * R, R)

    @pl.when(l == 0)
    def _fc0_and_seed():
        h = jnp.maximum(
            jnp.dot(x_ref[...], w0_ref[...],
                    preferred_element_type=jnp.float32) + b0_ref[...], 0.0)
        h0_ref[rows, :] = h
        ha_ref[rows, :] = h1_ref[...]

    def _conv(src_ref, dst_ref):
        lf = l.astype(jnp.float32) + 1.0
        hi = jnp.dot(adj8_ref[...], src_ref[...],
                     preferred_element_type=jnp.float32)
        hn = _layer_math(lf, hi, h0_ref[rows, :], cw_ref[0])
        dst_ref[rows, :] = hn.astype(_F8)

        @pl.when(l == L - 1)
        def _fc1():
            o = jnp.dot(hn, w1_ref[...],
                        preferred_element_type=jnp.float32) + b1_ref[...]
            m = jnp.max(o, axis=1, keepdims=True)
            s = o - m
            out_ref[...] = s - jnp.log(
                jnp.sum(jnp.exp(s), axis=1, keepdims=True))

    @pl.when((l > 0) & (l % 2 == 1))
    def _odd():
        _conv(ha_ref, hb_ref)

    @pl.when((l > 0) & (l % 2 == 0))
    def _even():
        _conv(hb_ref, ha_ref)


def kernel(x, adj, conv_weights, W_fc0, b_fc0, W_fc1, b_fc1):
    N, D = x.shape
    H = W_fc0.shape[1]
    L = conv_weights.shape[0]
    C = W_fc1.shape[1]
    R1 = next(c for c in (400, 200, 80, 40, 16, 8, N) if N % c == 0)
    R2 = next(c for c in (1000, 400, 200, 80, 40, 16, 8, N) if N % c == 0)

    b0r = b_fc0.reshape(1, H)
    b1r = b_fc1.reshape(1, C)

    adj8, h1 = pl.pallas_call(
        functools.partial(_call1_kernel, R=R1),
        grid=(2, N // R1),
        in_specs=[
            pl.BlockSpec((R1, D), lambda l, r: (jnp.where(l == 0, r, 0), 0)),
            pl.BlockSpec((R1, N), lambda l, r: (jnp.where(l == 0, 0, r), 0)),
            pl.BlockSpec((1, H, H), lambda l, r: (0, 0, 0)),
            pl.BlockSpec((D, H), lambda l, r: (0, 0)),
            pl.BlockSpec((1, H), lambda l, r: (0, 0)),
        ],
        out_specs=[
            pl.BlockSpec((R1, N), lambda l, r: (jnp.where(l == 0, 0, r), 0)),
            pl.BlockSpec((R1, H), lambda l, r: (jnp.where(l == 0, 0, r), 0)),
        ],
        out_shape=[
            jax.ShapeDtypeStruct((N, N), _F8),
            jax.ShapeDtypeStruct((N, H), _F8),
        ],
        scratch_shapes=[
            pltpu.VMEM((N, H), jnp.float32),
            pltpu.VMEM((N, H), jnp.bfloat16),
        ],
        compiler_params=pltpu.CompilerParams(
            dimension_semantics=("arbitrary", "arbitrary")),
    )(x, adj, conv_weights, W_fc0, b0r)

    out = pl.pallas_call(
        functools.partial(_call2_kernel, R=R2, L=L),
        grid=(L, N // R2),
        in_specs=[
            pl.BlockSpec((R2, D), lambda l, r: (jnp.where(l == 0, r, 0), 0)),
            pl.BlockSpec((R2, H), lambda l, r: (jnp.where(l == 0, r, 0), 0)),
            pl.BlockSpec((R2, N), lambda l, r: (jnp.where(l == 0, 0, r), 0)),
            pl.BlockSpec((1, H, H), lambda l, r: (l, 0, 0)),
            pl.BlockSpec((D, H), lambda l, r: (0, 0)),
            pl.BlockSpec((1, H), lambda l, r: (0, 0)),
            pl.BlockSpec((H, C), lambda l, r: (0, 0)),
            pl.BlockSpec((1, C), lambda l, r: (0, 0)),
        ],
        out_specs=pl.BlockSpec((R2, C), lambda l, r: (r, 0)),
        out_shape=jax.ShapeDtypeStruct((N, C), jnp.float32),
        scratch_shapes=[
            pltpu.VMEM((N, H), _F8),
            pltpu.VMEM((N, H), _F8),
            pltpu.VMEM((N, H), jnp.float32),
        ],
        compiler_params=pltpu.CompilerParams(
            dimension_semantics=("arbitrary", "arbitrary")),
    )(x, h1, adj8, conv_weights, W_fc0, b0r, W_fc1, b1r)
    return out
